# baseline (speedup 1.0000x reference)
import functools

import jax
import jax.numpy as jnp
from jax.experimental import pallas as pl
from jax.experimental.pallas import tpu as pltpu

_NLAB = 160


def _gcn_body(K, nsub, nlab, lab_ref, x_ref, W1_ref, b1_ref, W2_ref, b2_ref,
              W3_ref, b3_ref, gamma_ref, beta_ref, out_ref, acc, cnt):
    t = pl.program_id(0)
    D = x_ref.shape[1]
    for sub in range(nsub):
        _gcn_block(K, nlab, t * nsub + sub, sub * K, lab_ref, x_ref,
                   W1_ref, b1_ref, W2_ref, b2_ref, W3_ref, b3_ref,
                   gamma_ref, beta_ref, out_ref, acc, cnt)


def _gcn_block(K, nlab, gb, row0, lab_ref, x_ref, W1_ref, b1_ref, W2_ref,
               b2_ref, W3_ref, b3_ref, gamma_ref, beta_ref, out_ref,
               acc, cnt):
    D = x_ref.shape[1]

    lab = lab_ref[0, 0, pl.ds(row0, K)]
    lcol = lab[:, None]
    onehot = (lcol == jax.lax.broadcasted_iota(jnp.int32, (K, nlab), 1)
              ).astype(jnp.float32)
    onehotT = (jax.lax.broadcasted_iota(jnp.int32, (nlab, K), 0)
               == lab[None, :]).astype(jnp.float32)
    row_ge_col = (jax.lax.broadcasted_iota(jnp.int32, (K, K), 0)
                  >= jax.lax.broadcasted_iota(jnp.int32, (K, K), 1))
    M = jnp.where((lcol == lab[None, :]) & row_ge_col, 1.0, 0.0)

    live = gb > 0
    acc0 = jnp.where(live, acc[...], 0.0)
    cnt0 = jnp.where(live, cnt[...], 0.0)

    hi = jnp.floor(cnt0 * (1.0 / 256.0))
    lo = cnt0 - 256.0 * hi
    HL = jnp.concatenate([hi, lo], axis=1)
    Pc = jnp.dot(onehot, HL, preferred_element_type=jnp.float32)
    P = jnp.dot(onehot, acc0, preferred_element_type=jnp.float32)
    deg = (jnp.sum(M, axis=1)[:, None]
           + 256.0 * Pc[:, :D] + Pc[:, D:])
    dinv = jax.lax.rsqrt(deg)
    cnt[...] = cnt0 + jnp.sum(onehotT, axis=1)[:, None]

    def layer(inp, W_ref, b_ref, prior):
        h = jnp.dot(inp, W_ref[...], preferred_element_type=jnp.float32)
        g = h * dinv
        pref = jnp.dot(M, g, preferred_element_type=jnp.float32) + prior
        return jnp.maximum(pref * dinv + b_ref[...], 0.0), g

    h1, g1 = layer(x_ref[pl.ds(row0, K), :], W1_ref, b1_ref, P[:, :D])
    h2, g2 = layer(h1, W2_ref, b2_ref, P[:, D:2 * D])
    h3, g3 = layer(h2, W3_ref, b3_ref, P[:, 2 * D:3 * D])
    G = jnp.concatenate([g1, g2, g3], axis=1)
    acc[...] = acc0 + jnp.dot(onehotT, G,
                              preferred_element_type=jnp.float32)

    mean = jnp.mean(h3, axis=1, keepdims=True)
    c = h3 - mean
    var = jnp.mean(c * c, axis=1, keepdims=True)
    out_ref[pl.ds(row0, K), :] = (c * jax.lax.rsqrt(var + 1e-5)
                                  * gamma_ref[...] + beta_ref[...])


def _build(N, D, K, nsub, interpret=False):
    KB = K * nsub
    nsteps = N // KB
    full = lambda shape: pl.BlockSpec(shape, lambda t: (0,) * len(shape))
    return pl.pallas_call(
        functools.partial(_gcn_body, K, nsub, _NLAB),
        grid=(nsteps,),
        in_specs=[
            pl.BlockSpec((1, 1, KB), lambda t: (t, 0, 0)),
            pl.BlockSpec((KB, D), lambda t: (t, 0)),
            full((D, D)), full((1, D)),
            full((D, D)), full((1, D)),
            full((D, D)), full((1, D)),
            full((1, D)), full((1, D)),
        ],
        out_specs=pl.BlockSpec((KB, D), lambda t: (t, 0)),
        out_shape=jax.ShapeDtypeStruct((N, D), jnp.float32),
        scratch_shapes=[
            pltpu.VMEM((_NLAB, 3 * D), jnp.float32),
            pltpu.VMEM((_NLAB, D), jnp.float32),
        ],
        interpret=interpret,
    )


def _run(x, idx_TCR, W1, b1, W2, b2, W3, b3, gamma, beta, interpret=False):
    N, D = x.shape
    K = next(k for k in (200, 80, 40, 16, 8, 1) if N % k == 0)
    nsub = next(s for s in (10, 5, 4, 3, 2, 1) if N % (K * s) == 0)
    lab3 = idx_TCR.astype(jnp.int32).reshape(N // (K * nsub), 1, K * nsub)
    r1 = lambda v: v.reshape(1, D)
    return _build(N, D, K, nsub, interpret)(
        lab3, x, W1, r1(b1), W2, r1(b2), W3, r1(b3), r1(gamma), r1(beta))


def kernel(x, idx_TCR, W1, b1, W2, b2, W3, b3, gamma, beta):
    return _run(x, idx_TCR, W1, b1, W2, b2, W3, b3, gamma, beta)


# speedup vs baseline: 1.3658x; 1.3658x over previous
import functools

import jax
import jax.numpy as jnp
from jax.experimental import pallas as pl
from jax.experimental.pallas import tpu as pltpu

_NLAB = 160


def _gcn_body(K, nsub, nlab, lab_ref, x_ref, W1_ref, b1_ref, W2_ref, b2_ref,
              W3_ref, b3_ref, gamma_ref, beta_ref, out_ref, acc, cnt):
    t = pl.program_id(0)
    D = x_ref.shape[1]
    for sub in range(nsub):
        _gcn_block(K, nlab, t * nsub + sub, sub * K, lab_ref, x_ref,
                   W1_ref, b1_ref, W2_ref, b2_ref, W3_ref, b3_ref,
                   gamma_ref, beta_ref, out_ref, acc, cnt)


def _gcn_block(K, nlab, gb, row0, lab_ref, x_ref, W1_ref, b1_ref, W2_ref,
               b2_ref, W3_ref, b3_ref, gamma_ref, beta_ref, out_ref,
               acc, cnt):
    D = x_ref.shape[1]

    lab = lab_ref[0, 0, pl.ds(row0, K)]
    lcol = lab[:, None]
    onehot = (lcol == jax.lax.broadcasted_iota(jnp.int32, (K, nlab), 1)
              ).astype(jnp.float32)
    onehotT = (jax.lax.broadcasted_iota(jnp.int32, (nlab, K), 0)
               == lab[None, :]).astype(jnp.float32)
    row_ge_col = (jax.lax.broadcasted_iota(jnp.int32, (K, K), 0)
                  >= jax.lax.broadcasted_iota(jnp.int32, (K, K), 1))
    M = jnp.where((lcol == lab[None, :]) & row_ge_col, 1.0, 0.0)

    live = gb > 0
    acc0 = jnp.where(live, acc[...], 0.0)
    cnt0 = jnp.where(live, cnt[...], 0.0)

    hi = jnp.floor(cnt0 * (1.0 / 256.0))
    lo = cnt0 - 256.0 * hi
    HL = jnp.concatenate([hi, lo], axis=1)
    Pc = jnp.dot(onehot, HL, preferred_element_type=jnp.float32)
    P = jnp.dot(onehot, acc0, preferred_element_type=jnp.float32)
    deg = (jnp.sum(M, axis=1)[:, None]
           + 256.0 * Pc[:, :D] + Pc[:, D:])
    dinv = jax.lax.rsqrt(deg)
    cnt[...] = cnt0 + jnp.sum(onehotT, axis=1)[:, None]

    def layer(inp, W_ref, b_ref, prior):
        h = jnp.dot(inp, W_ref[...], preferred_element_type=jnp.float32)
        g = h * dinv
        pref = jnp.dot(M, g, preferred_element_type=jnp.float32) + prior
        return jnp.maximum(pref * dinv + b_ref[...], 0.0), g

    h1, g1 = layer(x_ref[pl.ds(row0, K), :], W1_ref, b1_ref, P[:, :D])
    h2, g2 = layer(h1, W2_ref, b2_ref, P[:, D:2 * D])
    h3, g3 = layer(h2, W3_ref, b3_ref, P[:, 2 * D:3 * D])
    G = jnp.concatenate([g1, g2, g3], axis=1)
    acc[...] = acc0 + jnp.dot(onehotT, G,
                              preferred_element_type=jnp.float32)

    mean = jnp.mean(h3, axis=1, keepdims=True)
    c = h3 - mean
    var = jnp.mean(c * c, axis=1, keepdims=True)
    out_ref[pl.ds(row0, K), :] = (c * jax.lax.rsqrt(var + 1e-5)
                                  * gamma_ref[...] + beta_ref[...])


def _build(N, D, K, nsub, interpret=False):
    KB = K * nsub
    nsteps = N // KB
    full = lambda shape: pl.BlockSpec(shape, lambda t: (0,) * len(shape))
    return pl.pallas_call(
        functools.partial(_gcn_body, K, nsub, _NLAB),
        grid=(nsteps,),
        in_specs=[
            pl.BlockSpec((1, 1, KB), lambda t: (t, 0, 0)),
            pl.BlockSpec((KB, D), lambda t: (t, 0)),
            full((D, D)), full((1, D)),
            full((D, D)), full((1, D)),
            full((D, D)), full((1, D)),
            full((1, D)), full((1, D)),
        ],
        out_specs=pl.BlockSpec((KB, D), lambda t: (t, 0)),
        out_shape=jax.ShapeDtypeStruct((N, D), jnp.float32),
        scratch_shapes=[
            pltpu.VMEM((_NLAB, 3 * D), jnp.float32),
            pltpu.VMEM((_NLAB, D), jnp.float32),
        ],
        interpret=interpret,
    )


def _run(x, idx_TCR, W1, b1, W2, b2, W3, b3, gamma, beta, interpret=False):
    N, D = x.shape
    K = next(k for k in (400, 200, 80, 40, 16, 8, 1) if N % k == 0)
    nsub = next(s for s in (5, 4, 3, 2, 1) if N % (K * s) == 0)
    lab3 = idx_TCR.astype(jnp.int32).reshape(N // (K * nsub), 1, K * nsub)
    r1 = lambda v: v.reshape(1, D)
    return _build(N, D, K, nsub, interpret)(
        lab3, x, W1, r1(b1), W2, r1(b2), W3, r1(b3), r1(gamma), r1(beta))


def kernel(x, idx_TCR, W1, b1, W2, b2, W3, b3, gamma, beta):
    return _run(x, idx_TCR, W1, b1, W2, b2, W3, b3, gamma, beta)
